# baseline (speedup 1.0000x reference)
import functools

import jax
import jax.numpy as jnp
from jax import lax
from jax.experimental import pallas as pl
from jax.experimental.pallas import tpu as pltpu
from jax.experimental.pallas import tpu_sc as plsc

N = 10000
E = 320000
D_IN = 128
H = 64

NC = 2
NS = 16
NW = NC * NS

EPW = E // NW
DCH = 80
DNCH = EPW // DCH
N_PAD = 10240
DPW = N_PAD // NS

NSL = H // 4
NT = N_PAD
EHALF = E // 2
CE = 2000
NCE = EHALF // CE
NG = CE // 16

_MESH = plsc.VectorSubcoreMesh(core_axis_name="c", subcore_axis_name="s")


def _zero_vec():
    return jnp.zeros((16,), jnp.float32)


@functools.partial(
    pl.kernel,
    out_type=jax.ShapeDtypeStruct((NC, N_PAD), jnp.float32),
    mesh=_MESH,
    scratch_types=[
        pltpu.VMEM((DNCH, DCH), jnp.int32),
        pltpu.VMEM((DNCH, DCH), jnp.float32),
        pltpu.VMEM((DPW,), jnp.float32),
        pltpu.VMEM_SHARED((N_PAD,), jnp.float32),
        pltpu.SemaphoreType.DMA,
        pltpu.SemaphoreType.DMA,
    ],
)
def _deg_kernel(dst_hbm, w_hbm, out_hbm, dst2d, w2d, zbuf, deg_sh, sm0, sm1):
    c = lax.axis_index("c")
    s = lax.axis_index("s")
    wid = c * NS + s

    cp0 = pltpu.async_copy(dst_hbm.at[wid], dst2d, sm0)
    cp1 = pltpu.async_copy(w_hbm.at[wid], w2d, sm1)

    def zfill(j, _):
        zbuf[pl.ds(j * 16, 16)] = _zero_vec()
        return 0

    lax.fori_loop(0, DPW // 16, zfill, 0)
    pltpu.sync_copy(zbuf, deg_sh.at[pl.ds(s * DPW, DPW)])
    cp0.wait()
    cp1.wait()
    plsc.subcore_barrier()

    sems = (sm0, sm1)

    def s_issue(i, b):
        pltpu.async_copy(w2d.at[i], deg_sh.at[dst2d.at[i]], sems[b], add=True)

    def s_wait(i, b):
        pltpu.make_async_copy(w2d.at[i], deg_sh.at[dst2d.at[i]],
                              sems[b]).wait()

    s_issue(0, 0)
    s_issue(1, 1)

    def pair(k, _):
        i0 = 2 * k
        s_wait(i0 - 2, 0)
        s_issue(i0, 0)
        s_wait(i0 - 1, 1)
        s_issue(i0 + 1, 1)
        return 0

    lax.fori_loop(1, (DNCH - 1) // 2, pair, 0)
    s_wait(DNCH - 3, 0)
    s_issue(DNCH - 1, 0)
    s_wait(DNCH - 2, 1)
    s_wait(DNCH - 1, 0)
    plsc.subcore_barrier()
    pltpu.sync_copy(deg_sh.at[pl.ds(s * DPW, DPW)],
                    out_hbm.at[c, pl.ds(s * DPW, DPW)])


@functools.partial(
    pl.kernel,
    out_type=jax.ShapeDtypeStruct((NW, 4, NT), jnp.float32),
    mesh=_MESH,
    scratch_types=[
        pltpu.VMEM((4, NT), jnp.float32),
        pltpu.VMEM((4, NT), jnp.float32),
        pltpu.VMEM((CE,), jnp.int32),
        pltpu.VMEM((CE,), jnp.int32),
        pltpu.VMEM((CE,), jnp.int32),
        pltpu.VMEM((CE,), jnp.int32),
        pltpu.VMEM((CE,), jnp.float32),
        pltpu.VMEM((CE,), jnp.float32),
        pltpu.SemaphoreType.DMA,
        pltpu.SemaphoreType.DMA,
        pltpu.SemaphoreType.DMA,
    ],
    compiler_params=pltpu.CompilerParams(needs_layout_passes=False,
                                         use_tc_tiling_on_sc=False),
)
def _edge_kernel(src_hbm, dst_hbm, w_hbm, yt_hbm, out_hbm,
                 y_sl, acc, s0, s1, d0, d1, w0, w1, lm0, lm1, ly):
    c = lax.axis_index("c")
    s = lax.axis_index("s")
    wid = c * NS + s
    sid = wid // 2
    half = wid % 2
    ebase = half * EHALF

    cp_y = pltpu.async_copy(yt_hbm.at[pl.ds(sid * 4, 4), :], y_sl, ly)

    srcb = (s0, s1)
    dstb = (d0, d1)
    wb = (w0, w1)
    sems = (lm0, lm1)

    def l_issue(i, b):
        base = ebase + i * CE
        pltpu.async_copy(src_hbm.at[pl.ds(base, CE)], srcb[b], sems[b])
        pltpu.async_copy(dst_hbm.at[pl.ds(base, CE)], dstb[b], sems[b])
        pltpu.async_copy(w_hbm.at[pl.ds(base, CE)], wb[b], sems[b])

    def l_wait(i, b):
        base = ebase + i * CE
        pltpu.make_async_copy(src_hbm.at[pl.ds(base, CE)], srcb[b],
                              sems[b]).wait()
        pltpu.make_async_copy(dst_hbm.at[pl.ds(base, CE)], dstb[b],
                              sems[b]).wait()
        pltpu.make_async_copy(w_hbm.at[pl.ds(base, CE)], wb[b],
                              sems[b]).wait()

    l_issue(0, 0)
    l_issue(1, 1)

    def zfill(k, _):
        j = k // (NT // 16)
        q = k % (NT // 16)
        acc[j, pl.ds(q * 16, 16)] = _zero_vec()
        return 0

    lax.fori_loop(0, 4 * (NT // 16), zfill, 0)
    cp_y.wait()

    def process(b):
        sb, db, wvb = srcb[b], dstb[b], wb[b]

        def grp(g, _):
            sv = sb[pl.ds(g * 16, 16)]
            dv = db[pl.ds(g * 16, 16)]
            wv = wvb[pl.ds(g * 16, 16)]
            for j in range(4):
                gj = plsc.load_gather(y_sl.at[j], [sv])
                plsc.addupdate_scatter(acc.at[j], [dv], gj * wv)
            return 0

        lax.fori_loop(0, NG, grp, 0)

    def pair(k, _):
        i0 = 2 * k
        l_wait(i0, 0)
        process(0)
        l_issue(i0 + 2, 0)
        l_wait(i0 + 1, 1)
        process(1)
        l_issue(i0 + 3, 1)
        return 0

    lax.fori_loop(0, NCE // 2 - 1, pair, 0)
    l_wait(NCE - 2, 0)
    process(0)
    l_wait(NCE - 1, 1)
    process(1)
    pltpu.sync_copy(acc, out_hbm.at[wid])


_BR = 1024


def _dense1_body(x_ref, w1_ref, degp_ref, yt_ref, dis_ref):
    xw = jnp.dot(x_ref[...], w1_ref[...], preferred_element_type=jnp.float32)
    deg = 1.0 + degp_ref[0:1, :] + degp_ref[1:2, :]
    dis = lax.rsqrt(deg)
    yt_ref[...] = dis * xw.T
    dis_ref[...] = dis


def _dense2_body(p_ref, yt_ref, dis_ref, w2t_ref, b1_ref, y2t_ref):
    dis = dis_ref[...]
    acc = (p_ref[:, 0] + p_ref[:, 1]).reshape(H, _BR)
    ht = jnp.maximum(dis * (acc + yt_ref[...]) + b1_ref[...], 0.0)
    y2t_ref[...] = dis * jnp.dot(w2t_ref[...], ht,
                                 preferred_element_type=jnp.float32)


def _dense3_body(p_ref, yt_ref, dis_ref, wlt_ref, b2_ref, bl_ref, z_ref):
    dis = dis_ref[...]
    acc = (p_ref[:, 0] + p_ref[:, 1]).reshape(H, _BR)
    ht = jnp.maximum(dis * (acc + yt_ref[...]) + b2_ref[...], 0.0)
    logit = jnp.dot(wlt_ref[...], ht, preferred_element_type=jnp.float32)
    z_ref[...] = jax.nn.sigmoid(logit + bl_ref[...])


def _dense1(X, W1, degp):
    return pl.pallas_call(
        _dense1_body,
        grid=(NT // _BR,),
        in_specs=[
            pl.BlockSpec((_BR, D_IN), lambda i: (i, 0)),
            pl.BlockSpec((D_IN, H), lambda i: (0, 0)),
            pl.BlockSpec((NC, _BR), lambda i: (0, i)),
        ],
        out_specs=[
            pl.BlockSpec((H, _BR), lambda i: (0, i)),
            pl.BlockSpec((1, _BR), lambda i: (0, i)),
        ],
        out_shape=[
            jax.ShapeDtypeStruct((H, NT), jnp.float32),
            jax.ShapeDtypeStruct((1, NT), jnp.float32),
        ],
    )(X, W1, degp)


def _dense2(p, yt, dis, W2T, b1c):
    return pl.pallas_call(
        _dense2_body,
        grid=(NT // _BR,),
        in_specs=[
            pl.BlockSpec((NSL, 2, 4, _BR), lambda i: (0, 0, 0, i)),
            pl.BlockSpec((H, _BR), lambda i: (0, i)),
            pl.BlockSpec((1, _BR), lambda i: (0, i)),
            pl.BlockSpec((H, H), lambda i: (0, 0)),
            pl.BlockSpec((H, 1), lambda i: (0, 0)),
        ],
        out_specs=pl.BlockSpec((H, _BR), lambda i: (0, i)),
        out_shape=jax.ShapeDtypeStruct((H, NT), jnp.float32),
    )(p, yt, dis, W2T, b1c)


def _dense3(p, yt, dis, WlT, b2c, blc):
    return pl.pallas_call(
        _dense3_body,
        grid=(NT // _BR,),
        in_specs=[
            pl.BlockSpec((NSL, 2, 4, _BR), lambda i: (0, 0, 0, i)),
            pl.BlockSpec((H, _BR), lambda i: (0, i)),
            pl.BlockSpec((1, _BR), lambda i: (0, i)),
            pl.BlockSpec((1, H), lambda i: (0, 0)),
            pl.BlockSpec((H, 1), lambda i: (0, 0)),
            pl.BlockSpec((1, 1), lambda i: (0, 0)),
        ],
        out_specs=pl.BlockSpec((1, _BR), lambda i: (0, i)),
        out_shape=jax.ShapeDtypeStruct((1, NT), jnp.float32),
    )(p, yt, dis, WlT, b2c, blc)


def kernel(X, edge_index, edge_weight, W1, b1, W2, b2, Wl, bl):
    src = edge_index[0]
    dst = edge_index[1]
    dstr = dst.reshape(NW, DNCH, DCH)
    ewr = edge_weight.reshape(NW, DNCH, DCH)

    degp = _deg_kernel(dstr, ewr)
    Xp = jnp.pad(X, ((0, NT - N), (0, 0)))
    y1t, dis = _dense1(Xp, W1, degp)

    p1 = _edge_kernel(src, dst, edge_weight, y1t).reshape(NSL, 2, 4, NT)
    y2t = _dense2(p1, y1t, dis, W2.T, b1.reshape(H, 1))

    p2 = _edge_kernel(src, dst, edge_weight, y2t).reshape(NSL, 2, 4, NT)
    zt = _dense3(p2, y2t, dis, Wl.T, b2.reshape(H, 1), bl.reshape(1, 1))
    return zt[0, :N]
